# baseline (speedup 1.0000x reference)
import jax
import jax.numpy as jnp
from jax.experimental import pallas as pl
from jax.experimental.pallas import tpu as pltpu

V = 256
D = 512
L = 4
H = 8
DH = D // H
B = 32
T = 512
NT = 64
K = 8
DT = 32
N = B * T
R = 512
NBLK = N // R
LN_EPS = 1e-5


def _nrm(h):
    m = h.mean(-1, keepdims=True)
    d = h - m
    v = (d * d).mean(-1, keepdims=True)
    return d / jnp.sqrt(v + LN_EPS)


def _embed_kernel(x_ref, emb_ref, pos_ref, o_ref):
    ids = x_ref[0, 0]
    onehot = (ids[:, None] == jax.lax.broadcasted_iota(jnp.int32, (T, V), 1))
    o_ref[0] = onehot.astype(jnp.float32) @ emb_ref[...] + pos_ref[...]


def _attn_kernel(h_ref, w_ref, bias_ref, wo_ref, bo_ref, mask_ref, o_ref):
    h = h_ref[...]
    hn = _nrm(h)
    qkv = hn @ w_ref[...] + bias_ref[...]
    madd = mask_ref[...]
    cols = []
    for hh in range(H):
        q = qkv[:, hh * DH:(hh + 1) * DH]
        k = qkv[:, D + hh * DH:D + (hh + 1) * DH]
        v = qkv[:, 2 * D + hh * DH:2 * D + (hh + 1) * DH]
        z = jax.lax.dot_general(q, k, (((1,), (1,)), ((), ()))) + madd
        mx = z.max(-1, keepdims=True)
        e = jnp.exp(z - mx)
        sm = e.sum(-1, keepdims=True)
        cols.append((e @ v) / sm)
    attn = jnp.concatenate(cols, axis=-1)
    o_ref[...] = h + attn @ wo_ref[...] + bo_ref[...]


def _ffn_kernel(h_ref, wr_ref, br_ref, exp_ref,
                w1_ref, b1_ref, w2_ref, b2_ref,
                o_ref, imp_ref, load_ref):
    h1 = h_ref[...]
    dn2 = _nrm(h1)
    rlog = dn2 @ wr_ref[...] + br_ref[...]
    m0 = rlog.max(-1, keepdims=True)
    ex = jnp.exp(rlog - m0)
    col = jax.lax.broadcasted_iota(jnp.int32, (R, NT), 1)
    work = rlog
    gates_u = jnp.zeros_like(rlog)
    for _ in range(K):
        cm = work.max(-1, keepdims=True)
        eq = work == cm
        fidx = jnp.where(eq, col, NT).min(-1, keepdims=True)
        first = col == fidx
        gates_u = gates_u + jnp.where(first, ex, 0.0)
        work = jnp.where(first, -jnp.inf, work)
    gates = gates_u / gates_u.sum(-1, keepdims=True)
    hidden = jnp.maximum(dn2 @ w1_ref[...] + b1_ref[...], 0.0)
    gate_exp = gates @ exp_ref[...]
    ffn = (hidden * gate_exp) @ w2_ref[...] + b2_ref[...]
    o_ref[...] = h1 + ffn
    sm_full = ex.sum(-1, keepdims=True)
    probs = ex / sm_full
    imp_ref[0] = probs.sum(0, keepdims=True)
    load_ref[0] = (gates > 0).astype(jnp.float32).sum(0, keepdims=True)


def _head_kernel(h_ref, w_ref, bias_ref, o_ref):
    o_ref[...] = _nrm(h_ref[...]) @ w_ref[...] + bias_ref[...]


def _row2(v):
    return v.reshape(1, -1)


@jax.jit
def _forward(x, params):
    x3 = x.reshape(B, 1, T).astype(jnp.int32)
    h = pl.pallas_call(
        _embed_kernel,
        grid=(B,),
        in_specs=[
            pl.BlockSpec((1, 1, T), lambda b: (b, 0, 0)),
            pl.BlockSpec((V, D), lambda b: (0, 0)),
            pl.BlockSpec((T, D), lambda b: (0, 0)),
        ],
        out_specs=pl.BlockSpec((1, T, D), lambda b: (b, 0, 0)),
        out_shape=jax.ShapeDtypeStruct((B, T, D), jnp.float32),
        compiler_params=pltpu.CompilerParams(
            dimension_semantics=("parallel",)),
    )(x3, params['embedding'], params['pos_embedding'][:T]).reshape(N, D)

    ri = jnp.arange(T, dtype=jnp.int32)
    mask_add = jnp.where(ri[:, None] >= ri[None, :], 0.0, -1e9
                         ).astype(jnp.float32)
    tile_of_col = jnp.arange(NT * DT, dtype=jnp.int32) // DT
    expand = (tile_of_col[None, :] ==
              jnp.arange(NT, dtype=jnp.int32)[:, None]).astype(jnp.float32)

    total_aux = jnp.float32(0.0)
    for lp in params['layers']:
        wqkv_eff = lp['ln1_s'][:, None] * lp['wqkv']
        bqkv_eff = lp['ln1_b'] @ lp['wqkv'] + lp['bqkv']
        qscale = jnp.concatenate([
            jnp.full((D,), 0.125, jnp.float32),
            jnp.ones((2 * D,), jnp.float32)])
        wqkv_eff = wqkv_eff * qscale[None, :]
        bqkv_eff = bqkv_eff * qscale
        wr_eff = lp['ln2_s'][:, None] * lp['wr']
        br_eff = lp['ln2_b'] @ lp['wr']
        w1_eff = lp['ln2_s'][:, None] * lp['w1']
        b1_eff = lp['ln2_b'] @ lp['w1'] + lp['b1']

        h = pl.pallas_call(
            _attn_kernel,
            grid=(B,),
            in_specs=[
                pl.BlockSpec((T, D), lambda i: (i, 0)),
                pl.BlockSpec((D, 3 * D), lambda i: (0, 0)),
                pl.BlockSpec((1, 3 * D), lambda i: (0, 0)),
                pl.BlockSpec((D, D), lambda i: (0, 0)),
                pl.BlockSpec((1, D), lambda i: (0, 0)),
                pl.BlockSpec((T, T), lambda i: (0, 0)),
            ],
            out_specs=pl.BlockSpec((T, D), lambda i: (i, 0)),
            out_shape=jax.ShapeDtypeStruct((N, D), jnp.float32),
            compiler_params=pltpu.CompilerParams(
                dimension_semantics=("parallel",)),
        )(h, wqkv_eff, _row2(bqkv_eff), lp['wo'], _row2(lp['bo']), mask_add)

        h, imp, load = pl.pallas_call(
            _ffn_kernel,
            grid=(NBLK,),
            in_specs=[
                pl.BlockSpec((R, D), lambda i: (i, 0)),
                pl.BlockSpec((D, NT), lambda i: (0, 0)),
                pl.BlockSpec((1, NT), lambda i: (0, 0)),
                pl.BlockSpec((NT, NT * DT), lambda i: (0, 0)),
                pl.BlockSpec((D, NT * DT), lambda i: (0, 0)),
                pl.BlockSpec((1, NT * DT), lambda i: (0, 0)),
                pl.BlockSpec((NT * DT, D), lambda i: (0, 0)),
                pl.BlockSpec((1, D), lambda i: (0, 0)),
            ],
            out_specs=[
                pl.BlockSpec((R, D), lambda i: (i, 0)),
                pl.BlockSpec((1, 1, NT), lambda i: (i, 0, 0)),
                pl.BlockSpec((1, 1, NT), lambda i: (i, 0, 0)),
            ],
            out_shape=[
                jax.ShapeDtypeStruct((N, D), jnp.float32),
                jax.ShapeDtypeStruct((NBLK, 1, NT), jnp.float32),
                jax.ShapeDtypeStruct((NBLK, 1, NT), jnp.float32),
            ],
            compiler_params=pltpu.CompilerParams(
                dimension_semantics=("parallel",)),
        )(h, wr_eff, _row2(br_eff), expand,
          w1_eff, _row2(b1_eff), lp['w2'].reshape(NT * DT, D),
          _row2(lp['b2']))
        total_aux = total_aux + NT * jnp.sum(
            (imp.sum((0, 1)) / N) * (load.sum((0, 1)) / N))

    head_w_eff = params['lnf_s'][:, None] * params['head_w']
    head_b_eff = params['lnf_b'] @ params['head_w'] + params['head_b']
    logits = pl.pallas_call(
        _head_kernel,
        grid=(NBLK,),
        in_specs=[
            pl.BlockSpec((R, D), lambda i: (i, 0)),
            pl.BlockSpec((D, V), lambda i: (0, 0)),
            pl.BlockSpec((1, V), lambda i: (0, 0)),
        ],
        out_specs=pl.BlockSpec((R, V), lambda i: (i, 0)),
        out_shape=jax.ShapeDtypeStruct((N, V), jnp.float32),
        compiler_params=pltpu.CompilerParams(
            dimension_semantics=("parallel",)),
    )(h, head_w_eff, _row2(head_b_eff)).reshape(B, T, V)

    return logits, total_aux


def kernel(x, params):
    return _forward(x, params)
